# Initial kernel scaffold; baseline (speedup 1.0000x reference)
import jax
import jax.numpy as jnp
from jax.experimental import pallas as pl


def kernel(x, enc_W, enc_b, dec_W, dec_b):
    raise NotImplementedError("write your pallas kernel here")


# speedup vs baseline: 1.3250x; 1.3250x over previous
import functools

import jax
import jax.numpy as jnp
from jax.experimental import pallas as pl
from jax.experimental.pallas import tpu as pltpu

DIM = 2048
HID = 16384
K = 16
N = 32
BH = 2048
NBLK = HID // BH


def _encode_body(x_ref, db_ref, ew_ref, eb_ref, h_ref):
    xc = x_ref[...] - db_ref[...]
    h = jax.lax.dot_general(xc, ew_ref[...], (((1,), (1,)), ((), ())),
                            preferred_element_type=jnp.float32)
    h_ref[...] = h + eb_ref[...]


def _topk_body(h_ref, hs_ref):
    h = h_ref[...]
    neg = jnp.finfo(jnp.float32).min
    col = jax.lax.broadcasted_iota(jnp.int32, (N, HID), 1)
    work = h
    keep = jnp.zeros((N, HID), dtype=jnp.bool_)
    for _ in range(K):
        m = jnp.max(work, axis=1, keepdims=True)
        is_m = work == m
        first = jnp.min(jnp.where(is_m, col, HID), axis=1, keepdims=True)
        sel = col == first
        keep = jnp.logical_or(keep, sel)
        work = jnp.where(sel, neg, work)
    hs_ref[...] = jnp.maximum(jnp.where(keep, h, 0.0), 0.0)


def _decode_body(hs_ref, dw_ref, db_ref, out_ref):
    i = pl.program_id(0)
    part = jax.lax.dot_general(hs_ref[...], dw_ref[...], (((1,), (1,)), ((), ())),
                               preferred_element_type=jnp.float32)

    @pl.when(i == 0)
    def _init():
        out_ref[...] = part + db_ref[...]

    @pl.when(i != 0)
    def _acc():
        out_ref[...] += part


def kernel(x, enc_W, enc_b, dec_W, dec_b):
    h = pl.pallas_call(
        _encode_body,
        grid=(NBLK,),
        in_specs=[
            pl.BlockSpec((N, DIM), lambda i: (0, 0)),
            pl.BlockSpec((DIM,), lambda i: (0,)),
            pl.BlockSpec((BH, DIM), lambda i: (i, 0)),
            pl.BlockSpec((BH,), lambda i: (i,)),
        ],
        out_specs=pl.BlockSpec((N, BH), lambda i: (0, i)),
        out_shape=jax.ShapeDtypeStruct((N, HID), jnp.float32),
    )(x, dec_b, enc_W, enc_b)

    hs = pl.pallas_call(
        _topk_body,
        in_specs=[pl.BlockSpec((N, HID), lambda: (0, 0))],
        out_specs=pl.BlockSpec((N, HID), lambda: (0, 0)),
        out_shape=jax.ShapeDtypeStruct((N, HID), jnp.float32),
    )(h)

    out = pl.pallas_call(
        _decode_body,
        grid=(NBLK,),
        in_specs=[
            pl.BlockSpec((N, BH), lambda i: (0, i)),
            pl.BlockSpec((DIM, BH), lambda i: (0, i)),
            pl.BlockSpec((DIM,), lambda i: (0,)),
        ],
        out_specs=pl.BlockSpec((N, DIM), lambda i: (0, 0)),
        out_shape=jax.ShapeDtypeStruct((N, DIM), jnp.float32),
    )(hs, dec_W, dec_b)
    return out
